# baseline (device time: 281856 ns/iter reference)
import jax
import jax.numpy as jnp
from jax import lax
from jax.experimental import pallas as pl
from jax.experimental.pallas import tpu as pltpu

N_DEV = 16
M = 4096
N = 2048
CH = M // N_DEV
SUB = 4
SCH = CH // SUB

WIRE = jnp.bfloat16
OUT_DTYPE = jnp.bfloat16

_MESH = pl.DeviceIdType.MESH


def kernel(x, w_mat):
    x = x.astype(jnp.bfloat16)
    w = w_mat.astype(jnp.bfloat16)

    def body(x_ref, w_ref, out_ref, buf, qbuf, amax_buf,
             rs_send, rs_recv, ag_send, ag_recv, ax_send, ax_recv):
        p = lax.axis_index("i")
        right = lax.rem(p + 1, N_DEV)
        left = lax.rem(p + N_DEV - 1, N_DEV)

        def partial(c):
            xs = x_ref[pl.ds(c * CH, CH), :]
            return jnp.dot(xs, w_ref[...], preferred_element_type=jnp.float32)

        def rs_rdma(h, s):
            sl = pl.ds(s * SCH, SCH)
            return pltpu.make_async_remote_copy(
                src_ref=buf.at[h, sl, :],
                dst_ref=buf.at[h + 1, sl, :],
                send_sem=rs_send.at[h, s],
                recv_sem=rs_recv.at[h, s],
                device_id=(right,),
                device_id_type=_MESH,
            )

        bsem = pltpu.get_barrier_semaphore()
        pl.semaphore_signal(bsem, inc=1, device_id=(left,), device_id_type=_MESH)
        pl.semaphore_signal(bsem, inc=1, device_id=(right,), device_id_type=_MESH)
        pl.semaphore_wait(bsem, 2)

        buf[0, :, :] = partial(p).astype(WIRE)
        descs = [rs_rdma(0, s) for s in range(SUB)]
        for dd in descs:
            dd.start()
        r_subs = []
        for h in range(N_DEV - 1):
            c = lax.rem(p + N_DEV - h - 1, N_DEV)
            pc = partial(c)
            if h < N_DEV - 2:
                nxt = []
                for s in range(SUB):
                    descs[s].wait()
                    sl = slice(s * SCH, (s + 1) * SCH)
                    buf[h + 1, sl, :] = (
                        buf[h + 1, sl, :].astype(jnp.float32) + pc[sl, :]
                    ).astype(WIRE)
                    nd = rs_rdma(h + 1, s)
                    nd.start()
                    nxt.append(nd)
                descs = nxt
            else:
                for s in range(SUB):
                    descs[s].wait()
                    sl = slice(s * SCH, (s + 1) * SCH)
                    r_subs.append(jnp.maximum(
                        buf[h + 1, sl, :].astype(jnp.float32) + pc[sl, :], 0.0
                    ))

        o = lax.rem(p + 1, N_DEV)
        amax_local = jnp.max(jnp.stack([jnp.max(rs) for rs in r_subs]))

        amax_buf[p, :, :] = jnp.full((8, 128), amax_local, jnp.float32)
        sends = []
        for d in range(1, N_DEV):
            t = lax.rem(p + d, N_DEV)
            rd = pltpu.make_async_remote_copy(
                src_ref=amax_buf.at[p],
                dst_ref=amax_buf.at[p],
                send_sem=ax_send.at[d - 1],
                recv_sem=ax_recv.at[d - 1],
                device_id=(t,),
                device_id_type=_MESH,
            )
            rd.start()
            sends.append(rd)
        for d in range(1, N_DEV):
            s = lax.rem(p + N_DEV - d, N_DEV)
            wd = pltpu.make_async_remote_copy(
                src_ref=amax_buf.at[p],
                dst_ref=amax_buf.at[s],
                send_sem=ax_send.at[d - 1],
                recv_sem=ax_recv.at[d - 1],
                device_id=(s,),
                device_id_type=_MESH,
            )
            wd.wait_recv()
        for rd in sends:
            rd.wait_send()

        amax_g = jnp.max(amax_buf[...])
        scale = amax_g / 127.0
        inv_scale = 127.0 / amax_g

        def quantize(v):
            return jnp.clip(
                lax.round(v * inv_scale, lax.RoundingMethod.TO_NEAREST_EVEN),
                0.0, 127.0,
            ).astype(jnp.int8)

        for s in range(SUB):
            sl = slice(s * SCH, (s + 1) * SCH)
            qs = quantize(r_subs[s])
            qbuf[o, sl, :] = qs
            out_ref[pl.ds(o * CH + s * SCH, SCH), :] = (
                qs.astype(jnp.float32) * scale
            ).astype(OUT_DTYPE)

        def ag_rdma(g, chunk, s):
            sl = pl.ds(s * SCH, SCH)
            return pltpu.make_async_remote_copy(
                src_ref=qbuf.at[chunk, sl, :],
                dst_ref=qbuf.at[chunk, sl, :],
                send_sem=ag_send.at[g, s],
                recv_sem=ag_recv.at[g, s],
                device_id=(right,),
                device_id_type=_MESH,
            )

        descs = [ag_rdma(0, o, s) for s in range(SUB)]
        for dd in descs:
            dd.start()
        for g in range(N_DEV - 1):
            c_r = lax.rem(o + N_DEV - g - 1, N_DEV)
            if g < N_DEV - 2:
                nxt = []
                for s in range(SUB):
                    descs[s].wait()
                    nd = ag_rdma(g + 1, c_r, s)
                    nd.start()
                    nxt.append(nd)
                descs = nxt
            else:
                for dd in descs:
                    dd.wait()
            out_ref[pl.ds(c_r * CH, CH), :] = (
                qbuf[c_r, :, :].astype(jnp.float32) * scale
            ).astype(OUT_DTYPE)

    try:
        cparams = pltpu.CompilerParams(collective_id=0)
    except AttributeError:
        cparams = pltpu.TPUCompilerParams(collective_id=0)

    return pl.pallas_call(
        body,
        out_shape=jax.ShapeDtypeStruct((M, N), OUT_DTYPE),
        in_specs=[
            pl.BlockSpec(memory_space=pltpu.VMEM),
            pl.BlockSpec(memory_space=pltpu.VMEM),
        ],
        out_specs=pl.BlockSpec(memory_space=pltpu.VMEM),
        scratch_shapes=[
            pltpu.VMEM((N_DEV, CH, N), WIRE),
            pltpu.VMEM((N_DEV, CH, N), jnp.int8),
            pltpu.VMEM((N_DEV, 8, 128), jnp.float32),
            pltpu.SemaphoreType.DMA((N_DEV - 1, SUB)),
            pltpu.SemaphoreType.DMA((N_DEV - 1, SUB)),
            pltpu.SemaphoreType.DMA((N_DEV - 1, SUB)),
            pltpu.SemaphoreType.DMA((N_DEV - 1, SUB)),
            pltpu.SemaphoreType.DMA((N_DEV - 1,)),
            pltpu.SemaphoreType.DMA((N_DEV - 1,)),
        ],
        compiler_params=cparams,
    )(x, w)


# device time: 163290 ns/iter; 1.7261x vs baseline; 1.7261x over previous
import jax
import jax.numpy as jnp
from jax import lax
from jax.experimental import pallas as pl
from jax.experimental.pallas import tpu as pltpu

N_DEV = 16
HALF = N_DEV // 2
M = 4096
N = 2048
CH = M // N_DEV
SUB = 2
SCH = CH // SUB

WIRE = jnp.bfloat16
OUT_DTYPE = jnp.bfloat16

_MESH = pl.DeviceIdType.MESH


def kernel(x, w_mat):
    x = x.astype(jnp.bfloat16)
    w = w_mat.astype(jnp.bfloat16)

    def body(x_ref, w_ref, out_ref, mbuf, pbuf, qbuf, amax_buf,
             mrs_send, mrs_recv, prs_send, prs_recv,
             mag_send, mag_recv, pag_send, pag_recv, ax_send, ax_recv):
        p = lax.axis_index("i")
        right = lax.rem(p + 1, N_DEV)
        left = lax.rem(p + N_DEV - 1, N_DEV)

        def partial(c):
            xs = x_ref[pl.ds(c * CH, CH), :]
            return jnp.dot(xs, w_ref[...], preferred_element_type=jnp.float32)

        def stream_rdma(buf, k, s, send_sems, recv_sems, target):
            sl = pl.ds(s * SCH, SCH)
            return pltpu.make_async_remote_copy(
                src_ref=buf.at[k, sl, :],
                dst_ref=buf.at[k + 1, sl, :],
                send_sem=send_sems.at[k, s],
                recv_sem=recv_sems.at[k, s],
                device_id=(target,),
                device_id_type=_MESH,
            )

        bsem = pltpu.get_barrier_semaphore()
        pl.semaphore_signal(bsem, inc=1, device_id=(left,), device_id_type=_MESH)
        pl.semaphore_signal(bsem, inc=1, device_id=(right,), device_id_type=_MESH)
        pl.semaphore_wait(bsem, 2)

        mbuf[0, :, :] = partial(lax.rem(p + HALF, N_DEV)).astype(WIRE)
        pbuf[0, :, :] = partial(lax.rem(p + HALF - 1, N_DEV)).astype(WIRE)
        m_descs = [stream_rdma(mbuf, 0, s, mrs_send, mrs_recv, left)
                   for s in range(SUB)]
        p_descs = [stream_rdma(pbuf, 0, s, prs_send, prs_recv, right)
                   for s in range(SUB)]
        for dd in m_descs + p_descs:
            dd.start()

        for k in range(HALF):
            if k < HALF - 1:
                m_r = lax.rem(p + HALF + 1 + k, N_DEV)
                pc_m = partial(m_r)
            if k < HALF - 2:
                p_r = lax.rem(p + HALF - 2 - k, N_DEV)
                pc_p = partial(p_r)
            for s in range(SUB):
                m_descs[s].wait()
                sl = slice(s * SCH, (s + 1) * SCH)
                if k < HALF - 1:
                    mbuf[k + 1, sl, :] = (
                        mbuf[k + 1, sl, :].astype(jnp.float32) + pc_m[sl, :]
                    ).astype(WIRE)
                    nd = stream_rdma(mbuf, k + 1, s, mrs_send, mrs_recv, left)
                    nd.start()
                    m_descs[s] = nd
            if k < HALF - 1:
                for s in range(SUB):
                    p_descs[s].wait()
                    sl = slice(s * SCH, (s + 1) * SCH)
                    if k < HALF - 2:
                        pbuf[k + 1, sl, :] = (
                            pbuf[k + 1, sl, :].astype(jnp.float32) + pc_p[sl, :]
                        ).astype(WIRE)
                        nd = stream_rdma(pbuf, k + 1, s, prs_send, prs_recv, right)
                        nd.start()
                        p_descs[s] = nd

        pc_own = partial(p)
        r_subs = []
        for s in range(SUB):
            sl = slice(s * SCH, (s + 1) * SCH)
            r_subs.append(jnp.maximum(
                mbuf[HALF, sl, :].astype(jnp.float32)
                + pbuf[HALF - 1, sl, :].astype(jnp.float32)
                + pc_own[sl, :], 0.0))

        amax_local = jnp.max(jnp.stack([jnp.max(rs) for rs in r_subs]))

        amax_buf[p, :, :] = jnp.full((8, 128), amax_local, jnp.float32)
        sends = []
        for d in range(1, N_DEV):
            t = lax.rem(p + d, N_DEV)
            rd = pltpu.make_async_remote_copy(
                src_ref=amax_buf.at[p],
                dst_ref=amax_buf.at[p],
                send_sem=ax_send.at[d - 1],
                recv_sem=ax_recv.at[d - 1],
                device_id=(t,),
                device_id_type=_MESH,
            )
            rd.start()
            sends.append(rd)
        for d in range(1, N_DEV):
            sdev = lax.rem(p + N_DEV - d, N_DEV)
            wd = pltpu.make_async_remote_copy(
                src_ref=amax_buf.at[p],
                dst_ref=amax_buf.at[sdev],
                send_sem=ax_send.at[d - 1],
                recv_sem=ax_recv.at[d - 1],
                device_id=(sdev,),
                device_id_type=_MESH,
            )
            wd.wait_recv()
        for rd in sends:
            rd.wait_send()

        amax_g = jnp.max(amax_buf[...])
        scale = amax_g / 127.0
        inv_scale = 127.0 / amax_g

        def quantize(v):
            return jnp.clip(
                lax.round(v * inv_scale, lax.RoundingMethod.TO_NEAREST_EVEN),
                0.0, 127.0,
            ).astype(jnp.int8)

        for s in range(SUB):
            sl = slice(s * SCH, (s + 1) * SCH)
            qs = quantize(r_subs[s])
            qbuf[p, sl, :] = qs
            out_ref[pl.ds(p * CH + s * SCH, SCH), :] = (
                qs.astype(jnp.float32) * scale
            ).astype(OUT_DTYPE)

        def ag_rdma(sems_s, sems_r, k, chunk, s, target):
            sl = pl.ds(s * SCH, SCH)
            return pltpu.make_async_remote_copy(
                src_ref=qbuf.at[chunk, sl, :],
                dst_ref=qbuf.at[chunk, sl, :],
                send_sem=sems_s.at[k, s],
                recv_sem=sems_r.at[k, s],
                device_id=(target,),
                device_id_type=_MESH,
            )

        def dequant(c):
            out_ref[pl.ds(c * CH, CH), :] = (
                qbuf[c, :, :].astype(jnp.float32) * scale
            ).astype(OUT_DTYPE)

        ma = [ag_rdma(mag_send, mag_recv, 0, p, s, left) for s in range(SUB)]
        pa = [ag_rdma(pag_send, pag_recv, 0, p, s, right) for s in range(SUB)]
        for dd in ma + pa:
            dd.start()
        for k in range(HALF):
            m_c = lax.rem(p + 1 + k, N_DEV)
            for s in range(SUB):
                ma[s].wait()
                if k < HALF - 1:
                    nd = ag_rdma(mag_send, mag_recv, k + 1, m_c, s, left)
                    nd.start()
                    ma[s] = nd
            dequant(m_c)
            if k < HALF - 1:
                p_c = lax.rem(p + N_DEV - 1 - k, N_DEV)
                for s in range(SUB):
                    pa[s].wait()
                    if k < HALF - 2:
                        nd = ag_rdma(pag_send, pag_recv, k + 1, p_c, s, right)
                        nd.start()
                        pa[s] = nd
                dequant(p_c)

    try:
        cparams = pltpu.CompilerParams(collective_id=0)
    except AttributeError:
        cparams = pltpu.TPUCompilerParams(collective_id=0)

    return pl.pallas_call(
        body,
        out_shape=jax.ShapeDtypeStruct((M, N), OUT_DTYPE),
        in_specs=[
            pl.BlockSpec(memory_space=pltpu.VMEM),
            pl.BlockSpec(memory_space=pltpu.VMEM),
        ],
        out_specs=pl.BlockSpec(memory_space=pltpu.VMEM),
        scratch_shapes=[
            pltpu.VMEM((HALF + 1, CH, N), WIRE),
            pltpu.VMEM((HALF, CH, N), WIRE),
            pltpu.VMEM((N_DEV, CH, N), jnp.int8),
            pltpu.VMEM((N_DEV, 8, 128), jnp.float32),
            pltpu.SemaphoreType.DMA((HALF, SUB)),
            pltpu.SemaphoreType.DMA((HALF, SUB)),
            pltpu.SemaphoreType.DMA((HALF - 1, SUB)),
            pltpu.SemaphoreType.DMA((HALF - 1, SUB)),
            pltpu.SemaphoreType.DMA((HALF, SUB)),
            pltpu.SemaphoreType.DMA((HALF, SUB)),
            pltpu.SemaphoreType.DMA((HALF - 1, SUB)),
            pltpu.SemaphoreType.DMA((HALF - 1, SUB)),
            pltpu.SemaphoreType.DMA((N_DEV - 1,)),
            pltpu.SemaphoreType.DMA((N_DEV - 1,)),
        ],
        compiler_params=cparams,
    )(x, w)
